# baseline (device time: 57623 ns/iter reference)
import jax
import jax.numpy as jnp
from jax import lax
from jax.experimental import pallas as pl
from jax.experimental.pallas import tpu as pltpu

N_DEV = 8
N_HOPS = N_DEV - 1
N_SUB = 4


def _ring(q):
    return jnp.where(q < 4, q, 11 - q)


def kernel(x, dy):
    m, d = x.shape
    _, f = dy.shape
    ch = d // N_DEV
    hf = f // 2
    qf = hf // N_SUB

    x_bf = x.astype(jnp.bfloat16)

    def body(x_ref, dy_hbm, out_ref, dy_bf, dyf, tmp,
             send_cw, recv_cw, send_ccw, recv_ccw,
             copy_sems, ss_cw, rs_cw, ss_ccw, rs_ccw):
        j = lax.axis_index("i")
        r = _ring(j)
        succ = _ring((r + 1) % N_DEV)
        pred = _ring((r - 1) % N_DEV)

        streams = []
        for b in range(N_SUB):
            streams.append(dict(
                t=2 * b, send=send_cw, recv=recv_cw, ss=ss_cw, rs=rs_cw,
                tgt=succ, b=b, col=b * qf,
                q_send=lambda s: (r - 1 - s) % N_DEV,
                q_arr=lambda s: (r - 2 - s) % N_DEV,
            ))
            streams.append(dict(
                t=2 * b + 1, send=send_ccw, recv=recv_ccw, ss=ss_ccw,
                rs=rs_ccw, tgt=pred, b=b, col=hf + b * qf,
                q_send=lambda s: (r + 1 + s) % N_DEV,
                q_arr=lambda s: (r + 2 + s) % N_DEV,
            ))

        def tile_copy(k, slot):
            return pltpu.make_async_copy(
                dy_hbm.at[:, pl.ds(streams[k]["col"], qf)],
                dyf.at[slot],
                copy_sems.at[slot],
            )

        copies = {}
        for k in (0, 1):
            copies[k] = tile_copy(k, k)
            copies[k].start()

        barrier = pltpu.get_barrier_semaphore()
        for nbr in (succ, pred):
            pl.semaphore_signal(barrier, inc=1, device_id=(nbr,),
                                device_id_type=pl.DeviceIdType.MESH)
        pl.semaphore_wait(barrier, 2)

        def partial(q, lo):
            return lax.dot_general(
                x_ref[:, pl.ds(_ring(q) * ch, ch)],
                dy_bf[:, pl.ds(lo, qf)],
                (((0,), (0,)), ((), ())),
                preferred_element_type=jnp.float32,
            )

        def make_rdma(st, s):
            slot = s % 2
            return pltpu.make_async_remote_copy(
                src_ref=st["send"].at[slot, st["b"]],
                dst_ref=st["recv"].at[s, st["b"]],
                send_sem=st["ss"].at[slot, st["b"]],
                recv_sem=st["rs"].at[s, st["b"]],
                device_id=(st["tgt"],),
                device_id_type=pl.DeviceIdType.MESH,
            )

        inflight = {}
        for k, st in enumerate(streams):
            slot = k % 2
            copies[k].wait()
            dy_bf[:, pl.ds(st["col"], qf)] = dyf[slot].astype(jnp.bfloat16)
            if k + 2 < len(streams):
                copies[k + 2] = tile_copy(k + 2, slot)
                copies[k + 2].start()
            st["send"][0, st["b"], :, :] = partial(
                st["q_send"](0), st["col"]).astype(jnp.bfloat16)
            rdma = make_rdma(st, 0)
            rdma.start()
            inflight[(st["t"], 0)] = rdma
        for st in streams:
            tmp[st["t"], :, :] = partial(st["q_arr"](0), st["col"])

        for s in range(N_HOPS):
            for st in streams:
                t, b = st["t"], st["b"]
                inflight[(t, s)].wait_recv()
                acc = tmp[t, :, :] + st["recv"][s, b, :, :].astype(jnp.float32)
                if s < N_HOPS - 1:
                    if s >= 1:
                        inflight[(t, s - 1)].wait_send()
                    st["send"][(s + 1) % 2, b, :, :] = acc.astype(jnp.bfloat16)
                    rdma = make_rdma(st, s + 1)
                    rdma.start()
                    inflight[(t, s + 1)] = rdma
                    tmp[t, :, :] = partial(st["q_arr"](s + 1), st["col"])
                else:
                    out_ref[:, pl.ds(st["col"], qf)] = acc

        for st in streams:
            inflight[(st["t"], N_HOPS - 2)].wait_send()
            inflight[(st["t"], N_HOPS - 1)].wait_send()

    return pl.pallas_call(
        body,
        out_shape=jax.ShapeDtypeStruct((ch, f), jnp.float32),
        in_specs=[
            pl.BlockSpec(memory_space=pltpu.VMEM),
            pl.BlockSpec(memory_space=pl.ANY),
        ],
        out_specs=pl.BlockSpec(memory_space=pltpu.VMEM),
        scratch_shapes=[
            pltpu.VMEM((m, f), jnp.bfloat16),
            pltpu.VMEM((2, m, qf), jnp.float32),
            pltpu.VMEM((2 * N_SUB, ch, qf), jnp.float32),
            pltpu.VMEM((2, N_SUB, ch, qf), jnp.bfloat16),
            pltpu.VMEM((N_HOPS, N_SUB, ch, qf), jnp.bfloat16),
            pltpu.VMEM((2, N_SUB, ch, qf), jnp.bfloat16),
            pltpu.VMEM((N_HOPS, N_SUB, ch, qf), jnp.bfloat16),
            pltpu.SemaphoreType.DMA((2,)),
            pltpu.SemaphoreType.DMA((2, N_SUB)),
            pltpu.SemaphoreType.DMA((N_HOPS, N_SUB)),
            pltpu.SemaphoreType.DMA((2, N_SUB)),
            pltpu.SemaphoreType.DMA((N_HOPS, N_SUB)),
        ],
        compiler_params=pltpu.CompilerParams(collective_id=0),
    )(x_bf, dy)


# device time: 48872 ns/iter; 1.1791x vs baseline; 1.1791x over previous
import jax
import jax.numpy as jnp
from jax import lax
from jax.experimental import pallas as pl
from jax.experimental.pallas import tpu as pltpu

N_DEV = 8
ORD = ((0, 1, 2), (1, 2, 0), (2, 0, 1))
SLOT_BASE = (0, 4, 6)


def kernel(x, dy):
    m, d = x.shape
    _, f = dy.shape
    ch = d // N_DEV
    widths = (11 * ch, 11 * ch, 10 * ch)
    col0s = (0, widths[0], widths[0] + widths[1])

    x_bf = x.astype(jnp.bfloat16)
    dy_bf = dy.astype(jnp.bfloat16)

    def body(x_ref, dy_ref, out_ref, acc,
             recv0, recv1, recv2, ss0, rs0, ss1, rs1, ss2, rs2):
        recvs = (recv0, recv1, recv2)
        sss = (ss0, ss1, ss2)
        rss = (rs0, rs1, rs2)

        j = lax.axis_index("i")
        jj = j % 4
        c = [((jj == 1) | (jj == 2)).astype(jnp.int32),
             (jj >= 2).astype(jnp.int32),
             j // 4]

        def qid(qx, qy, qz):
            return qz * 4 + qy * 2 + (qx + qy) % 2

        partners = []
        for dd in range(3):
            p = list(c)
            p[dd] = 1 - p[dd]
            partners.append(qid(p[0], p[1], p[2]))

        def chunk(g, k, i, sent):
            d0, d1, d2 = ORD[g]
            bits = [None, None, None]
            if k == 0:
                bits[d0] = 1 - c[d0] if sent else c[d0]
                bits[d1] = (i >> 1) & 1
                bits[d2] = i & 1
            elif k == 1:
                bits[d0] = c[d0]
                bits[d1] = 1 - c[d1] if sent else c[d1]
                bits[d2] = i
            else:
                bits[d0] = c[d0]
                bits[d1] = c[d1]
                bits[d2] = 1 - c[d2] if sent else c[d2]
            return qid(bits[0], bits[1], bits[2])

        barrier = pltpu.get_barrier_semaphore()
        for dd in range(3):
            pl.semaphore_signal(barrier, inc=1, device_id=(partners[dd],),
                                device_id_type=pl.DeviceIdType.MESH)
        pl.semaphore_wait(barrier, 3)

        def partial(q, g):
            return lax.dot_general(
                x_ref[:, pl.ds(q * ch, ch)],
                dy_ref[:, pl.ds(col0s[g], widths[g])],
                (((0,), (0,)), ((), ())),
                preferred_element_type=jnp.float32,
            ).astype(jnp.bfloat16)

        def make_rdma(g, k, i):
            q = chunk(g, k, i, sent=True)
            slot = SLOT_BASE[k] + i
            return pltpu.make_async_remote_copy(
                src_ref=acc.at[pl.ds(q * ch, ch),
                               pl.ds(col0s[g], widths[g])],
                dst_ref=recvs[g].at[slot],
                send_sem=sss[g].at[slot],
                recv_sem=rss[g].at[slot],
                device_id=(partners[ORD[g][k]],),
                device_id_type=pl.DeviceIdType.MESH,
            )

        inflight = {}

        for i in range(4):
            for g in range(3):
                q = chunk(g, 0, i, sent=True)
                acc[pl.ds(q * ch, ch), pl.ds(col0s[g], widths[g])] = partial(q, g)
                rdma = make_rdma(g, 0, i)
                rdma.start()
                inflight[(g, 0, i)] = rdma
        for i in range(4):
            for g in range(3):
                q = chunk(g, 0, i, sent=False)
                acc[pl.ds(q * ch, ch), pl.ds(col0s[g], widths[g])] = partial(q, g)

        for k in range(3):
            for g in range(3):
                for i in range(4 >> k):
                    slot = SLOT_BASE[k] + i
                    inflight[(g, k, i)].wait_recv()
                    q = chunk(g, k, i, sent=False)
                    rows = pl.ds(q * ch, ch)
                    cols = pl.ds(col0s[g], widths[g])
                    summed = (acc[rows, cols].astype(jnp.float32)
                              + recvs[g][slot].astype(jnp.float32))
                    if k < 2:
                        acc[rows, cols] = summed.astype(jnp.bfloat16)
                    else:
                        out_ref[:, cols] = summed
            if k < 2:
                for g in range(3):
                    for i in range(4 >> (k + 1)):
                        rdma = make_rdma(g, k + 1, i)
                        rdma.start()
                        inflight[(g, k + 1, i)] = rdma

        for k in range(3):
            for g in range(3):
                for i in range(4 >> k):
                    inflight[(g, k, i)].wait_send()

    return pl.pallas_call(
        body,
        out_shape=jax.ShapeDtypeStruct((ch, f), jnp.float32),
        in_specs=[
            pl.BlockSpec(memory_space=pltpu.VMEM),
            pl.BlockSpec(memory_space=pltpu.VMEM),
        ],
        out_specs=pl.BlockSpec(memory_space=pltpu.VMEM),
        scratch_shapes=[
            pltpu.VMEM((d, f), jnp.bfloat16),
            pltpu.VMEM((7, ch, widths[0]), jnp.bfloat16),
            pltpu.VMEM((7, ch, widths[1]), jnp.bfloat16),
            pltpu.VMEM((7, ch, widths[2]), jnp.bfloat16),
            pltpu.SemaphoreType.DMA((7,)),
            pltpu.SemaphoreType.DMA((7,)),
            pltpu.SemaphoreType.DMA((7,)),
            pltpu.SemaphoreType.DMA((7,)),
            pltpu.SemaphoreType.DMA((7,)),
            pltpu.SemaphoreType.DMA((7,)),
        ],
        compiler_params=pltpu.CompilerParams(collective_id=0),
    )(x_bf, dy_bf)
